# baseline (speedup 1.0000x reference)
import functools

import jax
import jax.numpy as jnp
from jax import lax
from jax.experimental import pallas as pl
from jax.experimental.pallas import tpu as pltpu
from jax.experimental.pallas import tpu_sc as plsc

TOP_K = 2
TM = 256


def _router_body(x_ref, wg_ref, logits_ref, ids_ref, w_ref, xb_ref):
    x = x_ref[...]
    xb_ref[...] = x.astype(jnp.bfloat16)
    logits = jax.lax.dot_general(
        x, wg_ref[...], (((1,), (0,)), ((), ())),
        preferred_element_type=jnp.float32)
    logits_ref[...] = logits
    e = logits.shape[1]
    lane = jax.lax.broadcasted_iota(jnp.int32, logits.shape, 1)
    m0 = jnp.max(logits, axis=1, keepdims=True)
    e0 = jnp.min(jnp.where(logits == m0, lane, e), axis=1, keepdims=True)
    l2 = jnp.where(lane == e0, -jnp.inf, logits)
    m1 = jnp.max(l2, axis=1, keepdims=True)
    e1 = jnp.min(jnp.where(l2 == m1, lane, e), axis=1, keepdims=True)
    w1 = 1.0 / (1.0 + jnp.exp(m0 - m1))
    ids_ref[...] = jnp.concatenate([e0, e1], axis=1)
    w_ref[...] = jnp.concatenate([1.0 - w1, w1], axis=1)


def _router(x2d, wg):
    t, d = x2d.shape
    e = wg.shape[1]
    bt = min(1024, t)
    return pl.pallas_call(
        _router_body,
        grid=(t // bt,),
        in_specs=[
            pl.BlockSpec((bt, d), lambda i: (i, 0)),
            pl.BlockSpec((d, e), lambda i: (0, 0)),
        ],
        out_specs=[
            pl.BlockSpec((bt, e), lambda i: (i, 0)),
            pl.BlockSpec((bt, TOP_K), lambda i: (i, 0)),
            pl.BlockSpec((bt, TOP_K), lambda i: (i, 0)),
            pl.BlockSpec((bt, d), lambda i: (i, 0)),
        ],
        out_shape=[
            jax.ShapeDtypeStruct((t, e), jnp.float32),
            jax.ShapeDtypeStruct((t, TOP_K), jnp.int32),
            jax.ShapeDtypeStruct((t, TOP_K), jnp.float32),
            jax.ShapeDtypeStruct((t, d), jnp.bfloat16),
        ],
    )(x2d, wg)


def _sc_dispatch(x_i32, src3):
    t, dw = x_i32.shape
    nw, nch, c = src3.shape
    mesh = plsc.VectorSubcoreMesh(core_axis_name="c", subcore_axis_name="s")

    @functools.partial(
        pl.kernel, mesh=mesh,
        out_type=jax.ShapeDtypeStruct((nw * nch * c, dw), jnp.int32),
        scratch_types=[
            pltpu.VMEM((c,), jnp.int32),
            pltpu.VMEM((c, dw), jnp.int32),
            pltpu.SemaphoreType.DMA,
        ],
    )
    def k(x_hbm, idx_hbm, out_hbm, idx_v, rows_v, sem):
        wid = lax.axis_index("s") * 2 + lax.axis_index("c")

        def body(j, carry):
            pltpu.sync_copy(idx_hbm.at[wid, j], idx_v)
            pltpu.async_copy(x_hbm.at[idx_v], rows_v, sem).wait()
            pltpu.sync_copy(rows_v, out_hbm.at[pl.ds((wid * nch + j) * c, c)])
            return carry

        lax.fori_loop(0, nch, body, 0)

    return k(x_i32, src3)


def _sc_combine(yw, p0_3, p1_3):
    a, d = yw.shape
    nw, nch, c = p0_3.shape
    t = nw * nch * c
    mesh = plsc.VectorSubcoreMesh(core_axis_name="c", subcore_axis_name="s")

    @functools.partial(
        pl.kernel, mesh=mesh,
        out_type=jax.ShapeDtypeStruct((t, d), jnp.float32),
        scratch_types=[
            pltpu.VMEM((c,), jnp.int32),
            pltpu.VMEM((c,), jnp.int32),
            pltpu.VMEM((c, d), jnp.float32),
            pltpu.VMEM((c, d), jnp.float32),
            pltpu.SemaphoreType.DMA,
            pltpu.SemaphoreType.DMA,
        ],
    )
    def k(y_hbm, p0_hbm, p1_hbm, out_hbm, i0_v, i1_v, r0_v, r1_v, s0, s1):
        wid = lax.axis_index("s") * 2 + lax.axis_index("c")

        def body(j, carry):
            pltpu.sync_copy(p0_hbm.at[wid, j], i0_v)
            pltpu.sync_copy(p1_hbm.at[wid, j], i1_v)
            cp0 = pltpu.async_copy(y_hbm.at[i0_v], r0_v, s0)
            cp1 = pltpu.async_copy(y_hbm.at[i1_v], r1_v, s1)
            cp0.wait()
            cp1.wait()

            def row(r, carry2):
                for kk in range(d // 16):
                    sl = pl.ds(kk * 16, 16)
                    r0_v[r, sl] = r0_v[r, sl] + r1_v[r, sl]
                return carry2

            lax.fori_loop(0, c, row, 0)
            pltpu.sync_copy(r0_v, out_hbm.at[pl.ds((wid * nch + j) * c, c)])
            return carry

        lax.fori_loop(0, nch, body, 0)

    return k(yw, p0_3, p1_3)


_GELU_A = 2.0 * 0.7978845608028654
_GELU_B = _GELU_A * 0.044715


def _moe_body(te_ref, x_ref, wi_ref, wv_ref, wo_ref, wt_ref, y_ref):
    x = x_ref[...]
    h = jax.lax.dot_general(
        x, wi_ref[0], (((1,), (0,)), ((), ())),
        preferred_element_type=jnp.float32)
    v = jax.lax.dot_general(
        x, wv_ref[0], (((1,), (0,)), ((), ())),
        preferred_element_type=jnp.float32)
    u = h * (_GELU_A + _GELU_B * (h * h))
    g = h / (1.0 + jnp.exp(-u)) * v
    y = jax.lax.dot_general(
        g.astype(x.dtype), wo_ref[0], (((1,), (0,)), ((), ())),
        preferred_element_type=jnp.float32)
    y_ref[...] = y * wt_ref[0]


def _moe_mlp(x_sorted, w_in, w_v, w_out, tile_expert, w_tile):
    a, d = x_sorted.shape
    e, _, f = w_in.shape
    nt = a // TM
    grid_spec = pltpu.PrefetchScalarGridSpec(
        num_scalar_prefetch=1,
        grid=(nt,),
        in_specs=[
            pl.BlockSpec((TM, d), lambda i, te: (i, 0)),
            pl.BlockSpec((1, d, f), lambda i, te: (te[i], 0, 0)),
            pl.BlockSpec((1, d, f), lambda i, te: (te[i], 0, 0)),
            pl.BlockSpec((1, f, d), lambda i, te: (te[i], 0, 0)),
            pl.BlockSpec((1, TM, 1), lambda i, te: (i, 0, 0)),
        ],
        out_specs=pl.BlockSpec((TM, d), lambda i, te: (i, 0)),
    )
    return pl.pallas_call(
        _moe_body,
        grid_spec=grid_spec,
        out_shape=jax.ShapeDtypeStruct((a, d), jnp.float32),
    )(tile_expert, x_sorted, w_in, w_v, w_out, w_tile)


def kernel(hidden_states, Wg, W_in, W_v, W_out):
    b, s, d = hidden_states.shape
    e = Wg.shape[1]
    t = b * s
    n_assign = t * TOP_K
    a = n_assign + e * TM
    nt = a // TM
    nw = 32

    x2d = hidden_states.reshape(t, d)
    logits, ids, w, xb = _router(x2d, Wg)

    ex = ids.reshape(-1)
    oh = (ex[:, None] == jnp.arange(e, dtype=jnp.int32)[None, :]).astype(jnp.int32)
    cum = jnp.cumsum(oh, axis=0)
    rank = jnp.take_along_axis(cum, ex[:, None].astype(jnp.int32), axis=1)[:, 0] - 1
    counts = cum[-1]
    padded = ((counts + TM - 1) // TM) * TM
    ends = jnp.cumsum(padded)
    base = ends - padded
    pos = (base[ex] + rank).astype(jnp.int32)
    src = jnp.zeros((a,), jnp.int32).at[pos].set(
        jnp.arange(n_assign, dtype=jnp.int32) // TOP_K)
    wsorted = jnp.zeros((a,), jnp.float32).at[pos].set(w.reshape(-1))
    tile_expert = jnp.clip(
        jnp.searchsorted(ends, jnp.arange(nt, dtype=jnp.int32) * TM,
                         side="right"),
        0, e - 1).astype(jnp.int32)

    x_i32 = lax.bitcast_convert_type(xb.reshape(t, d // 2, 2), jnp.int32)
    c_d = a // (nw * 6)
    xs_i32 = _sc_dispatch(x_i32, src.reshape(nw, 6, c_d))
    x_sorted = lax.bitcast_convert_type(xs_i32, jnp.bfloat16).reshape(a, d)

    yw = _moe_mlp(x_sorted, W_in.astype(jnp.bfloat16), W_v.astype(jnp.bfloat16),
                  W_out.astype(jnp.bfloat16), tile_expert,
                  wsorted.reshape(nt, TM, 1))

    pos2 = pos.reshape(t, TOP_K)
    c_c = t // (nw * 4)
    out = _sc_combine(yw, pos2[:, 0].reshape(nw, 4, c_c),
                      pos2[:, 1].reshape(nw, 4, c_c))
    return out.reshape(b, s, d), logits.reshape(b, s, e)


# speedup vs baseline: 1.5523x; 1.5523x over previous
import functools

import jax
import jax.numpy as jnp
from jax import lax
from jax.experimental import pallas as pl
from jax.experimental.pallas import tpu as pltpu
from jax.experimental.pallas import tpu_sc as plsc

TOP_K = 2
TM = 256


def _router_body(x_ref, wg_ref, logits_ref, ids_ref, w_ref):
    x = x_ref[...]
    logits = jax.lax.dot_general(
        x, wg_ref[...], (((1,), (0,)), ((), ())),
        preferred_element_type=jnp.float32)
    logits_ref[...] = logits
    e = logits.shape[1]
    lane = jax.lax.broadcasted_iota(jnp.int32, logits.shape, 1)
    m0 = jnp.max(logits, axis=1, keepdims=True)
    e0 = jnp.min(jnp.where(logits == m0, lane, e), axis=1, keepdims=True)
    l2 = jnp.where(lane == e0, -jnp.inf, logits)
    m1 = jnp.max(l2, axis=1, keepdims=True)
    e1 = jnp.min(jnp.where(l2 == m1, lane, e), axis=1, keepdims=True)
    w1 = 1.0 / (1.0 + jnp.exp(m0 - m1))
    ids_ref[...] = jnp.concatenate([e0, e1], axis=1)
    w_ref[...] = jnp.concatenate([1.0 - w1, w1], axis=1)


def _router(x2d, wg):
    t, d = x2d.shape
    e = wg.shape[1]
    bt = min(1024, t)
    return pl.pallas_call(
        _router_body,
        grid=(t // bt,),
        in_specs=[
            pl.BlockSpec((bt, d), lambda i: (i, 0)),
            pl.BlockSpec((d, e), lambda i: (0, 0)),
        ],
        out_specs=[
            pl.BlockSpec((bt, e), lambda i: (i, 0)),
            pl.BlockSpec((bt, TOP_K), lambda i: (i, 0)),
            pl.BlockSpec((bt, TOP_K), lambda i: (i, 0)),
        ],
        out_shape=[
            jax.ShapeDtypeStruct((t, e), jnp.float32),
            jax.ShapeDtypeStruct((t, TOP_K), jnp.int32),
            jax.ShapeDtypeStruct((t, TOP_K), jnp.float32),
        ],
    )(x2d, wg)


def _sc_dispatch(x_f32, src3):
    t, d = x_f32.shape
    nw, nch, c = src3.shape

    mesh = plsc.VectorSubcoreMesh(core_axis_name="c", subcore_axis_name="s")

    @functools.partial(
        pl.kernel, mesh=mesh,
        out_type=jax.ShapeDtypeStruct((nw * nch * c, d), jnp.float32),
        scratch_types=[
            pltpu.VMEM((nch, c), jnp.int32),
            pltpu.VMEM((c, d), jnp.float32),
            pltpu.VMEM((c, d), jnp.float32),
            pltpu.SemaphoreType.DMA((2,)),
            pltpu.SemaphoreType.DMA((2,)),
        ],
    )
    def k(x_hbm, idx_hbm, out_hbm, idx_v, r0, r1, gsem, osem):
        wid = lax.axis_index("s") * 2 + lax.axis_index("c")
        pltpu.sync_copy(idx_hbm.at[wid], idx_v)
        bufs = (r0, r1)

        def gather(j):
            return pltpu.async_copy(
                x_hbm.at[idx_v.at[j]], bufs[j % 2], gsem.at[j % 2])

        def put(j):
            return pltpu.async_copy(
                bufs[j % 2],
                out_hbm.at[pl.ds((wid * nch + j) * c, c)], osem.at[j % 2])

        gcp = {0: gather(0)}
        ocp = {}
        for j in range(nch):
            nxt = j + 1
            if nxt < nch:
                if nxt - 2 >= 0:
                    ocp[nxt - 2].wait()
                gcp[nxt] = gather(nxt)
            gcp[j].wait()
            ocp[j] = put(j)
        ocp[nch - 2].wait()
        ocp[nch - 1].wait()

    return k(x_f32, src3)


def _sc_combine(yw, p0_3, p1_3):
    a, d = yw.shape
    nw, nch, c = p0_3.shape
    t = nw * nch * c

    mesh = plsc.VectorSubcoreMesh(core_axis_name="c", subcore_axis_name="s")

    @functools.partial(
        pl.kernel, mesh=mesh,
        out_type=jax.ShapeDtypeStruct((t, d), jnp.float32),
        scratch_types=[
            pltpu.VMEM((nch, c), jnp.int32),
            pltpu.VMEM((nch, c), jnp.int32),
            pltpu.VMEM((c, d), jnp.float32),
            pltpu.VMEM((c, d), jnp.float32),
            pltpu.VMEM((c, d), jnp.float32),
            pltpu.VMEM((c, d), jnp.float32),
            pltpu.SemaphoreType.DMA((2,)),
            pltpu.SemaphoreType.DMA((2,)),
            pltpu.SemaphoreType.DMA((2,)),
        ],
    )
    def k(y_hbm, p0_hbm, p1_hbm, out_hbm, i0_v, i1_v,
          a0, a1, b0, b1, g0sem, g1sem, osem):
        wid = lax.axis_index("s") * 2 + lax.axis_index("c")
        pltpu.sync_copy(p0_hbm.at[wid], i0_v)
        pltpu.sync_copy(p1_hbm.at[wid], i1_v)
        abufs = (a0, a1)
        bbufs = (b0, b1)

        def gathers(j):
            return (
                pltpu.async_copy(y_hbm.at[i0_v.at[j]], abufs[j % 2],
                                 g0sem.at[j % 2]),
                pltpu.async_copy(y_hbm.at[i1_v.at[j]], bbufs[j % 2],
                                 g1sem.at[j % 2]),
            )

        def put(j):
            return pltpu.async_copy(
                abufs[j % 2],
                out_hbm.at[pl.ds((wid * nch + j) * c, c)], osem.at[j % 2])

        gcp = {0: gathers(0)}
        ocp = {}
        for j in range(nch):
            nxt = j + 1
            if nxt < nch:
                if nxt - 2 >= 0:
                    ocp[nxt - 2].wait()
                gcp[nxt] = gathers(nxt)
            gcp[j][0].wait()
            gcp[j][1].wait()
            ra, rb = abufs[j % 2], bbufs[j % 2]

            def row(r, carry, ra=ra, rb=rb):
                for kk in range(d // 16):
                    sl = pl.ds(kk * 16, 16)
                    plsc.addupdate(ra.at[r, sl], rb[r, sl])
                return carry

            lax.fori_loop(0, c, row, 0)
            ocp[j] = put(j)
        ocp[nch - 2].wait()
        ocp[nch - 1].wait()

    return k(yw, p0_3, p1_3)


_GELU_A = 2.0 * 0.7978845608028654
_GELU_B = _GELU_A * 0.044715


def _moe_body(te_ref, x_ref, wi_ref, wv_ref, wo_ref, wt_ref, y_ref):
    x = x_ref[...].astype(jnp.bfloat16)
    h = jax.lax.dot_general(
        x, wi_ref[0], (((1,), (0,)), ((), ())),
        preferred_element_type=jnp.float32)
    v = jax.lax.dot_general(
        x, wv_ref[0], (((1,), (0,)), ((), ())),
        preferred_element_type=jnp.float32)
    u = h * (_GELU_A + _GELU_B * (h * h))
    g = h / (1.0 + jnp.exp(-u)) * v
    y = jax.lax.dot_general(
        g.astype(jnp.bfloat16), wo_ref[0], (((1,), (0,)), ((), ())),
        preferred_element_type=jnp.float32)
    y_ref[...] = y * wt_ref[0]


def _moe_mlp(x_sorted, w_in, w_v, w_out, tile_expert, w_tile):
    a, d = x_sorted.shape
    e, _, f = w_in.shape
    nt = a // TM
    grid_spec = pltpu.PrefetchScalarGridSpec(
        num_scalar_prefetch=1,
        grid=(nt,),
        in_specs=[
            pl.BlockSpec((TM, d), lambda i, te: (i, 0)),
            pl.BlockSpec((1, d, f), lambda i, te: (te[i], 0, 0)),
            pl.BlockSpec((1, d, f), lambda i, te: (te[i], 0, 0)),
            pl.BlockSpec((1, f, d), lambda i, te: (te[i], 0, 0)),
            pl.BlockSpec((1, TM, 1), lambda i, te: (i, 0, 0)),
        ],
        out_specs=pl.BlockSpec((TM, d), lambda i, te: (i, 0)),
    )
    return pl.pallas_call(
        _moe_body,
        grid_spec=grid_spec,
        out_shape=jax.ShapeDtypeStruct((a, d), jnp.float32),
    )(tile_expert, x_sorted, w_in, w_v, w_out, w_tile)


def kernel(hidden_states, Wg, W_in, W_v, W_out):
    b, s, d = hidden_states.shape
    e = Wg.shape[1]
    t = b * s
    n_assign = t * TOP_K
    a = n_assign + e * TM
    nt = a // TM
    nw = 32

    x2d = hidden_states.reshape(t, d)
    logits, ids, w = _router(x2d, Wg)

    ex = ids.T.reshape(-1)
    oh = (ex[:, None] == jnp.arange(e, dtype=jnp.int32)[None, :]).astype(jnp.int32)
    cum = jnp.cumsum(oh, axis=0)
    rank = jnp.take_along_axis(cum, ex[:, None].astype(jnp.int32), axis=1)[:, 0] - 1
    counts = cum[-1]
    padded = ((counts + TM - 1) // TM) * TM
    ends = jnp.cumsum(padded)
    base = ends - padded
    pos = (base[ex] + rank).astype(jnp.int32)
    src = jnp.zeros((a,), jnp.int32).at[pos].set(
        jnp.arange(n_assign, dtype=jnp.int32) % t)
    wsorted = jnp.zeros((a,), jnp.float32).at[pos].set(w.T.reshape(-1))
    tile_expert = jnp.clip(
        jnp.searchsorted(ends, jnp.arange(nt, dtype=jnp.int32) * TM,
                         side="right"),
        0, e - 1).astype(jnp.int32)

    nch_d = 9
    x_sorted = _sc_dispatch(x2d, src.reshape(nw, nch_d, a // (nw * nch_d)))

    yw = _moe_mlp(x_sorted, W_in.astype(jnp.bfloat16), W_v.astype(jnp.bfloat16),
                  W_out.astype(jnp.bfloat16), tile_expert,
                  wsorted.reshape(nt, TM, 1))

    nch_c = 8
    c_c = t // (nw * nch_c)
    out = _sc_combine(yw, pos[:t].reshape(nw, nch_c, c_c),
                      pos[t:].reshape(nw, nch_c, c_c))
    return out.reshape(b, s, d), logits.reshape(b, s, e)
